# baseline (speedup 1.0000x reference)
import functools

import jax
import jax.numpy as jnp
from jax import lax
from jax.experimental import pallas as pl
from jax.experimental.pallas import tpu as pltpu
from jax.experimental.pallas import tpu_sc as plsc

NU = 10000
NC = 10000
E = 320000
EL = 100000
F = 128
H = 128

SC_CORES = 2
SC_TILES = 16

CHUNK = 128
E_PER_TILE = 20480
E_PAD = E_PER_TILE * SC_TILES
N_CHUNKS = E_PER_TILE // CHUNK
ROWS_PER_TILE = 632
N_PAD = ROWS_PER_TILE * SC_TILES
DUMMY_ROW = 10000

N_CHUNKS_CLS = 26
EL_PER_TILE = N_CHUNKS_CLS * CHUNK
EL_PAD = EL_PER_TILE * SC_CORES * SC_TILES

MB = 1000


def _dot_t(a, b):
    return lax.dot_general(a, b, (((1,), (1,)), ((), ())),
                           preferred_element_type=jnp.float32)


def _phase_a_body(x_ref, w1_ref, b1_ref, emb_ref, wl_ref, wr_ref, bl_ref,
                  y_ref, r_ref):
    t = _dot_t(x_ref[...], w1_ref[...]) + b1_ref[...] + emb_ref[...]
    y_ref[...] = _dot_t(t, wl_ref[...])
    r_ref[...] = _dot_t(t, wr_ref[...]) + bl_ref[...]


def _phase_a(x, w1, b1, emb, wl, wr, bl):
    grid = (NU // MB,)
    row = pl.BlockSpec((MB, F), lambda i: (i, 0))
    full = pl.BlockSpec((H, F), lambda i: (0, 0))
    vec = pl.BlockSpec((1, H), lambda i: (0, 0))
    return pl.pallas_call(
        _phase_a_body,
        grid=grid,
        in_specs=[row, full, vec, row, full, full, vec],
        out_specs=[row, row],
        out_shape=[jax.ShapeDtypeStruct((NU, H), jnp.float32)] * 2,
    )(x, w1, b1.reshape(1, H), emb, wl, wr, bl.reshape(1, H))


def _phase_c_body(s_ref, cnt_ref, r1_ref, wl_ref, wr_ref, bl_ref,
                  y2_ref, r2_ref):
    cnt = jnp.maximum(cnt_ref[...][:, 0:1], 1.0)
    h = jnp.maximum(s_ref[...] / cnt + r1_ref[...], 0.0)
    y2_ref[...] = _dot_t(h, wl_ref[...])
    r2_ref[...] = _dot_t(h, wr_ref[...]) + bl_ref[...]


def _phase_c(s, cnt, r1, wl, wr, bl):
    grid = (NU // MB,)
    row = pl.BlockSpec((MB, H), lambda i: (i, 0))
    crow = pl.BlockSpec((MB, H), lambda i: (i, 0))
    full = pl.BlockSpec((H, H), lambda i: (0, 0))
    vec = pl.BlockSpec((1, H), lambda i: (0, 0))
    return pl.pallas_call(
        _phase_c_body,
        grid=grid,
        in_specs=[row, crow, row, full, full, vec],
        out_specs=[row, row],
        out_shape=[jax.ShapeDtypeStruct((NU, H), jnp.float32)] * 2,
    )(s, cnt, r1, wl, wr, bl.reshape(1, H))


def _phase_e_body(s_ref, cnt_ref, r2_ref, o_ref):
    cnt = jnp.maximum(cnt_ref[...][:, 0:1], 1.0)
    o_ref[...] = s_ref[...] / cnt + r2_ref[...]


def _phase_e(s, cnt, r2):
    grid = (NU // MB,)
    row = pl.BlockSpec((MB, H), lambda i: (i, 0))
    crow = pl.BlockSpec((MB, H), lambda i: (i, 0))
    return pl.pallas_call(
        _phase_e_body,
        grid=grid,
        in_specs=[row, crow, row],
        out_specs=row,
        out_shape=jax.ShapeDtypeStruct((NU, H), jnp.float32),
    )(s, cnt, r2)


IDXB = 16
NBLK = N_CHUNKS // IDXB


def _make_seg_kernel():
    mesh = plsc.VectorSubcoreMesh(core_axis_name="c", subcore_axis_name="s")

    @functools.partial(
        pl.kernel,
        out_type=[jax.ShapeDtypeStruct((N_PAD, H), jnp.float32)] * 2,
        mesh=mesh,
        scratch_types=[
            pltpu.VMEM((IDXB, CHUNK), jnp.int32),
            pltpu.VMEM((IDXB, CHUNK), jnp.int32),
            pltpu.VMEM((CHUNK, H), jnp.float32),
            pltpu.VMEM((CHUNK, H), jnp.float32),
            pltpu.VMEM_SHARED((N_PAD, H), jnp.float32),
            pltpu.SemaphoreType.DMA,
            pltpu.SemaphoreType.DMA,
            pltpu.SemaphoreType.DMA,
            pltpu.SemaphoreType.DMA,
        ],
    )
    def seg(y_a, y_b, src_a, dst_a, src_b, dst_b, zacc,
            out_a, out_b, sidx_v, didx_v, rows0, rows1, acc_sh,
            gs0, gs1, ss0, ss1):
        core = lax.axis_index("c")
        tile = lax.axis_index("s")
        r0 = tile * ROWS_PER_TILE

        pltpu.sync_copy(zacc.at[pl.ds(r0, ROWS_PER_TILE)],
                        acc_sh.at[pl.ds(r0, ROWS_PER_TILE)])
        plsc.subcore_barrier()

        def loop(y_hbm, src2d, dst2d):
            for bi in range(NBLK):
                brow = tile * N_CHUNKS + bi * IDXB
                pltpu.sync_copy(src2d.at[pl.ds(brow, IDXB)], sidx_v)
                pltpu.sync_copy(dst2d.at[pl.ds(brow, IDXB)], didx_v)

                def pair(p, carry):
                    j0 = 2 * p
                    j1 = 2 * p + 1
                    g0 = pltpu.async_copy(y_hbm.at[sidx_v.at[j0]], rows0, gs0)
                    g1 = pltpu.async_copy(y_hbm.at[sidx_v.at[j1]], rows1, gs1)
                    g0.wait()
                    s0 = pltpu.async_copy(rows0, acc_sh.at[didx_v.at[j0]],
                                          ss0, add=True)
                    g1.wait()
                    s1 = pltpu.async_copy(rows1, acc_sh.at[didx_v.at[j1]],
                                          ss1, add=True)
                    s0.wait()
                    s1.wait()
                    return carry

                lax.fori_loop(0, IDXB // 2, pair, 0)

        @pl.when(core == 0)
        def _():
            loop(y_a, src_a, dst_a)

        @pl.when(core == 1)
        def _():
            loop(y_b, src_b, dst_b)

        plsc.subcore_barrier()

        def out_copy(s_out):
            ro = 0
            while ro < ROWS_PER_TILE:
                rn = min(CHUNK, ROWS_PER_TILE - ro)
                pltpu.sync_copy(acc_sh.at[pl.ds(r0 + ro, rn)],
                                rows0.at[pl.ds(0, rn)])
                pltpu.sync_copy(rows0.at[pl.ds(0, rn)],
                                s_out.at[pl.ds(r0 + ro, rn)])
                ro += rn

        @pl.when(core == 0)
        def _():
            out_copy(out_a)

        @pl.when(core == 1)
        def _():
            out_copy(out_b)

    return seg


_seg_kernel = _make_seg_kernel()


def _make_cnt_kernel():
    mesh = plsc.VectorSubcoreMesh(core_axis_name="c", subcore_axis_name="s")

    @functools.partial(
        pl.kernel,
        out_type=[jax.ShapeDtypeStruct((N_PAD, H), jnp.float32)] * 2,
        mesh=mesh,
        scratch_types=[
            pltpu.VMEM((IDXB, CHUNK), jnp.int32),
            pltpu.VMEM((CHUNK, H), jnp.float32),
            pltpu.VMEM_SHARED((N_PAD, H), jnp.float32),
            pltpu.SemaphoreType.DMA,
            pltpu.SemaphoreType.DMA,
        ],
    )
    def cnt(dst_a, dst_b, zacc, ones_hbm, out_a, out_b,
            didx_v, rows_v, acc_sh, ss0, ss1):
        core = lax.axis_index("c")
        tile = lax.axis_index("s")
        r0 = tile * ROWS_PER_TILE

        pltpu.sync_copy(zacc.at[pl.ds(r0, ROWS_PER_TILE)],
                        acc_sh.at[pl.ds(r0, ROWS_PER_TILE)])
        pltpu.sync_copy(ones_hbm, rows_v)
        plsc.subcore_barrier()

        def loop(dst2d):
            for bi in range(NBLK):
                brow = tile * N_CHUNKS + bi * IDXB
                pltpu.sync_copy(dst2d.at[pl.ds(brow, IDXB)], didx_v)

                def pair(p, carry):
                    s0 = pltpu.async_copy(rows_v, acc_sh.at[didx_v.at[2 * p]],
                                          ss0, add=True)
                    s1 = pltpu.async_copy(rows_v,
                                          acc_sh.at[didx_v.at[2 * p + 1]],
                                          ss1, add=True)
                    s0.wait()
                    s1.wait()
                    return carry

                lax.fori_loop(0, IDXB // 2, pair, 0)

        @pl.when(core == 0)
        def _():
            loop(dst_a)

        @pl.when(core == 1)
        def _():
            loop(dst_b)

        plsc.subcore_barrier()

        def out_copy(c_out):
            ro = 0
            while ro < ROWS_PER_TILE:
                rn = min(CHUNK, ROWS_PER_TILE - ro)
                pltpu.sync_copy(acc_sh.at[pl.ds(r0 + ro, rn)],
                                rows_v.at[pl.ds(0, rn)])
                pltpu.sync_copy(rows_v.at[pl.ds(0, rn)],
                                c_out.at[pl.ds(r0 + ro, rn)])
                ro += rn

        @pl.when(core == 0)
        def _():
            out_copy(out_a)

        @pl.when(core == 1)
        def _():
            out_copy(out_b)

    return cnt


_cnt_kernel = _make_cnt_kernel()


def _cls_gather_body(ou_hbm, oc_hbm, ia2d, ib2d, ga_out, gb_out,
                     iav, ibv, a0, a1, b0, b1,
                     ga0, ga1, gb0, gb1, wa0, wa1, wb0, wb1):
    core = lax.axis_index("c")
    tile = lax.axis_index("s")
    wid = core * SC_TILES + tile
    ebase = wid * EL_PER_TILE
    npair = N_CHUNKS_CLS // 2

    pltpu.sync_copy(ia2d.at[wid], iav)
    pltpu.sync_copy(ib2d.at[wid], ibv)

    pltpu.async_copy(ou_hbm.at[iav.at[0]], a0, ga0)
    pltpu.async_copy(oc_hbm.at[ibv.at[0]], b0, gb0)
    pltpu.async_copy(ou_hbm.at[iav.at[1]], a1, ga1)
    pltpu.async_copy(oc_hbm.at[ibv.at[1]], b1, gb1)

    def pair(p, carry):
        j0 = 2 * p
        j1 = 2 * p + 1
        o0 = ebase + j0 * CHUNK
        o1 = ebase + j1 * CHUNK
        pltpu.make_async_copy(ou_hbm.at[iav.at[j0]], a0, ga0).wait()
        w_a0 = pltpu.async_copy(a0, ga_out.at[pl.ds(o0, CHUNK)], wa0)
        pltpu.make_async_copy(oc_hbm.at[ibv.at[j0]], b0, gb0).wait()
        w_b0 = pltpu.async_copy(b0, gb_out.at[pl.ds(o0, CHUNK)], wb0)
        pltpu.make_async_copy(ou_hbm.at[iav.at[j1]], a1, ga1).wait()
        w_a1 = pltpu.async_copy(a1, ga_out.at[pl.ds(o1, CHUNK)], wa1)
        pltpu.make_async_copy(oc_hbm.at[ibv.at[j1]], b1, gb1).wait()
        w_b1 = pltpu.async_copy(b1, gb_out.at[pl.ds(o1, CHUNK)], wb1)

        @pl.when(p < npair - 1)
        def _():
            w_a0.wait()
            pltpu.async_copy(ou_hbm.at[iav.at[j0 + 2]], a0, ga0)
            w_b0.wait()
            pltpu.async_copy(oc_hbm.at[ibv.at[j0 + 2]], b0, gb0)
            w_a1.wait()
            pltpu.async_copy(ou_hbm.at[iav.at[j1 + 2]], a1, ga1)
            w_b1.wait()
            pltpu.async_copy(oc_hbm.at[ibv.at[j1 + 2]], b1, gb1)

        @pl.when(p == npair - 1)
        def _():
            w_a0.wait()
            w_b0.wait()
            w_a1.wait()
            w_b1.wait()

        return carry

    lax.fori_loop(0, npair, pair, 0)


_cls_gather = pl.kernel(
    _cls_gather_body,
    out_type=[jax.ShapeDtypeStruct((EL_PAD, H), jnp.float32)] * 2,
    mesh=plsc.VectorSubcoreMesh(core_axis_name="c", subcore_axis_name="s"),
    scratch_types=(
        [pltpu.VMEM((32, CHUNK), jnp.int32)] * 2 +
        [pltpu.VMEM((CHUNK, H), jnp.float32)] * 4 +
        [pltpu.SemaphoreType.DMA] * 8
    ),
)

CLS_MB = 1024


def _cls_dot_body(a_ref, b_ref, o_ref):
    o_ref[...] = jnp.sum(a_ref[...] * b_ref[...], axis=1, keepdims=True)


def _cls_dot(ga, gb):
    grid = (EL_PAD // CLS_MB,)
    row = pl.BlockSpec((CLS_MB, H), lambda i: (i, 0))
    out = pl.BlockSpec((CLS_MB, 1), lambda i: (i, 0))
    return pl.pallas_call(
        _cls_dot_body,
        grid=grid,
        in_specs=[row, row],
        out_specs=out,
        out_shape=jax.ShapeDtypeStruct((EL_PAD, 1), jnp.float32),
    )(ga, gb)


def _pad_edges(idx, n, pad_val):
    pad = jnp.full((n - idx.shape[0],), pad_val, jnp.int32)
    return jnp.concatenate([idx.astype(jnp.int32), pad])


def kernel(x_user, x_content, user_lin_w, user_lin_b, content_lin_w,
           content_lin_b, user_emb, content_emb,
           c1_uc_wl, c1_uc_bl, c1_uc_wr, c1_cu_wl, c1_cu_bl, c1_cu_wr,
           c2_uc_wl, c2_uc_bl, c2_uc_wr, c2_cu_wl, c2_cu_bl, c2_cu_wr,
           edge_index_uc, edge_index_cu, edge_label_index):
    n2d = E_PAD // CHUNK
    src_cu = _pad_edges(edge_index_cu[0], E_PAD, 0).reshape(n2d, CHUNK)
    dst_cu = _pad_edges(edge_index_cu[1], E_PAD, DUMMY_ROW).reshape(n2d, CHUNK)
    src_uc = _pad_edges(edge_index_uc[0], E_PAD, 0).reshape(n2d, CHUNK)
    dst_uc = _pad_edges(edge_index_uc[1], E_PAD, DUMMY_ROW).reshape(n2d, CHUNK)
    def _labels3d(idx):
        x = _pad_edges(idx, EL_PAD, 0).reshape(32, N_CHUNKS_CLS, CHUNK)
        return jnp.pad(x, ((0, 0), (0, 32 - N_CHUNKS_CLS), (0, 0)))

    la = _labels3d(edge_label_index[0])
    lb = _labels3d(edge_label_index[1])

    zacc = jnp.zeros((N_PAD, H), jnp.float32)
    ones_hbm = jnp.ones((CHUNK, H), jnp.float32)

    cnt_u, cnt_c = _cnt_kernel(dst_cu, dst_uc, zacc, ones_hbm)

    yu1, ru1 = _phase_a(x_user, user_lin_w, user_lin_b, user_emb,
                        c1_uc_wl, c1_cu_wr, c1_cu_bl)
    yc1, rc1 = _phase_a(x_content, content_lin_w, content_lin_b, content_emb,
                        c1_cu_wl, c1_uc_wr, c1_uc_bl)

    su1, sc1 = _seg_kernel(yc1, yu1, src_cu, dst_cu, src_uc, dst_uc, zacc)

    yu2, ru2 = _phase_c(su1, cnt_u, ru1, c2_uc_wl, c2_cu_wr, c2_cu_bl)
    yc2, rc2 = _phase_c(sc1, cnt_c, rc1, c2_cu_wl, c2_uc_wr, c2_uc_bl)

    su2, sc2 = _seg_kernel(yc2, yu2, src_cu, dst_cu, src_uc, dst_uc, zacc)

    ou = _phase_e(su2, cnt_u, ru2)
    oc = _phase_e(sc2, cnt_c, rc2)

    ga, gb = _cls_gather(ou, oc, la, lb)
    pred = _cls_dot(ga, gb)
    return pred[:EL, 0]


# speedup vs baseline: 2.1449x; 2.1449x over previous
import functools

import jax
import jax.numpy as jnp
from jax import lax
from jax.experimental import pallas as pl
from jax.experimental.pallas import tpu as pltpu
from jax.experimental.pallas import tpu_sc as plsc

NU = 10000
NC = 10000
E = 320000
EL = 100000
F = 128
H = 128

SC_CORES = 2
SC_TILES = 16

CHUNK = 128
SEGC = 64
E_PER_TILE = 20480
E_PAD = E_PER_TILE * SC_TILES
N_CHUNKS = E_PER_TILE // CHUNK
SEG_NCH = E_PER_TILE // SEGC
SEG_IDXB = 32
SEG_NBLK = SEG_NCH // SEG_IDXB
ROWS_PER_TILE = 632
N_PAD = ROWS_PER_TILE * SC_TILES
DUMMY_ROW = 10000

N_CHUNKS_CLS = 26
EL_PER_TILE = N_CHUNKS_CLS * CHUNK
EL_PAD = EL_PER_TILE * SC_CORES * SC_TILES

MB = 1000


def _dot_t(a, b):
    return lax.dot_general(a, b, (((1,), (1,)), ((), ())),
                           preferred_element_type=jnp.float32)


def _phase_a_body(x_ref, w1_ref, b1_ref, emb_ref, wl_ref, wr_ref, bl_ref,
                  y_ref, r_ref):
    t = _dot_t(x_ref[...], w1_ref[...]) + b1_ref[...] + emb_ref[...]
    y_ref[...] = _dot_t(t, wl_ref[...])
    r_ref[...] = _dot_t(t, wr_ref[...]) + bl_ref[...]


def _phase_a(x, w1, b1, emb, wl, wr, bl):
    grid = (NU // MB,)
    row = pl.BlockSpec((MB, F), lambda i: (i, 0))
    full = pl.BlockSpec((H, F), lambda i: (0, 0))
    vec = pl.BlockSpec((1, H), lambda i: (0, 0))
    return pl.pallas_call(
        _phase_a_body,
        grid=grid,
        in_specs=[row, full, vec, row, full, full, vec],
        out_specs=[row, row],
        out_shape=[jax.ShapeDtypeStruct((NU, H), jnp.float32)] * 2,
    )(x, w1, b1.reshape(1, H), emb, wl, wr, bl.reshape(1, H))


def _phase_c_body(s_ref, cnt_ref, r1_ref, wl_ref, wr_ref, bl_ref,
                  y2_ref, r2_ref):
    cnt = jnp.maximum(cnt_ref[...][:, 0:1], 1.0)
    h = jnp.maximum(s_ref[...] / cnt + r1_ref[...], 0.0)
    y2_ref[...] = _dot_t(h, wl_ref[...])
    r2_ref[...] = _dot_t(h, wr_ref[...]) + bl_ref[...]


def _phase_c(s, cnt, r1, wl, wr, bl):
    grid = (NU // MB,)
    row = pl.BlockSpec((MB, H), lambda i: (i, 0))
    crow = pl.BlockSpec((MB, H), lambda i: (i, 0))
    full = pl.BlockSpec((H, H), lambda i: (0, 0))
    vec = pl.BlockSpec((1, H), lambda i: (0, 0))
    return pl.pallas_call(
        _phase_c_body,
        grid=grid,
        in_specs=[row, crow, row, full, full, vec],
        out_specs=[row, row],
        out_shape=[jax.ShapeDtypeStruct((NU, H), jnp.float32)] * 2,
    )(s, cnt, r1, wl, wr, bl.reshape(1, H))


def _phase_e_body(s_ref, cnt_ref, r2_ref, o_ref):
    cnt = jnp.maximum(cnt_ref[...][:, 0:1], 1.0)
    o_ref[...] = s_ref[...] / cnt + r2_ref[...]


def _phase_e(s, cnt, r2):
    grid = (NU // MB,)
    row = pl.BlockSpec((MB, H), lambda i: (i, 0))
    crow = pl.BlockSpec((MB, H), lambda i: (i, 0))
    return pl.pallas_call(
        _phase_e_body,
        grid=grid,
        in_specs=[row, crow, row],
        out_specs=row,
        out_shape=jax.ShapeDtypeStruct((NU, H), jnp.float32),
    )(s, cnt, r2)


IDXB = 16
NBLK = N_CHUNKS // IDXB


def _make_seg_kernel():
    mesh = plsc.VectorSubcoreMesh(core_axis_name="c", subcore_axis_name="s")

    @functools.partial(
        pl.kernel,
        out_type=[jax.ShapeDtypeStruct((N_PAD, H), jnp.float32)] * 2,
        mesh=mesh,
        scratch_types=(
            [pltpu.VMEM((SEG_IDXB, SEGC), jnp.int32)] * 2 +
            [pltpu.VMEM((SEGC, H), jnp.float32)] * 4 +
            [pltpu.VMEM_SHARED((N_PAD, H), jnp.float32)] +
            [pltpu.SemaphoreType.DMA] * 8
        ),
    )
    def seg(y_a, y_b, src_a, dst_a, src_b, dst_b, zacc,
            out_a, out_b, sidx_v, didx_v, r0_, r1_, r2_, r3_, acc_sh,
            g0s, g1s, g2s, g3s, s0s, s1s, s2s, s3s):
        core = lax.axis_index("c")
        tile = lax.axis_index("s")
        r0 = tile * ROWS_PER_TILE

        pltpu.sync_copy(zacc.at[pl.ds(r0, ROWS_PER_TILE)],
                        acc_sh.at[pl.ds(r0, ROWS_PER_TILE)])
        plsc.subcore_barrier()

        bufs = [(r0_, g0s, s0s), (r1_, g1s, s1s),
                (r2_, g2s, s2s), (r3_, g3s, s3s)]

        def loop(y_hbm, src2d, dst2d):
            for bi in range(SEG_NBLK):
                brow = tile * SEG_NCH + bi * SEG_IDXB
                pltpu.sync_copy(src2d.at[pl.ds(brow, SEG_IDXB)], sidx_v)
                pltpu.sync_copy(dst2d.at[pl.ds(brow, SEG_IDXB)], didx_v)

                def quad(q, carry):
                    jb = 4 * q
                    gs = [pltpu.async_copy(y_hbm.at[sidx_v.at[jb + k]],
                                           bufs[k][0], bufs[k][1])
                          for k in range(4)]
                    ss = []
                    for k in range(4):
                        gs[k].wait()
                        ss.append(pltpu.async_copy(
                            bufs[k][0], acc_sh.at[didx_v.at[jb + k]],
                            bufs[k][2], add=True))
                    for k in range(4):
                        ss[k].wait()
                    return carry

                lax.fori_loop(0, SEG_IDXB // 4, quad, 0)

        @pl.when(core == 0)
        def _():
            loop(y_a, src_a, dst_a)

        @pl.when(core == 1)
        def _():
            loop(y_b, src_b, dst_b)

        plsc.subcore_barrier()

        def out_copy(s_out):
            ro = 0
            while ro < ROWS_PER_TILE:
                rn = min(SEGC, ROWS_PER_TILE - ro)
                pltpu.sync_copy(acc_sh.at[pl.ds(r0 + ro, rn)],
                                r0_.at[pl.ds(0, rn)])
                pltpu.sync_copy(r0_.at[pl.ds(0, rn)],
                                s_out.at[pl.ds(r0 + ro, rn)])
                ro += rn

        @pl.when(core == 0)
        def _():
            out_copy(out_a)

        @pl.when(core == 1)
        def _():
            out_copy(out_b)

    return seg


_seg_kernel = _make_seg_kernel()


def _make_cnt_kernel():
    mesh = plsc.VectorSubcoreMesh(core_axis_name="c", subcore_axis_name="s")

    @functools.partial(
        pl.kernel,
        out_type=[jax.ShapeDtypeStruct((N_PAD, H), jnp.float32)] * 2,
        mesh=mesh,
        scratch_types=[
            pltpu.VMEM((SEG_IDXB, SEGC), jnp.int32),
            pltpu.VMEM((CHUNK, H), jnp.float32),
            pltpu.VMEM_SHARED((N_PAD, H), jnp.float32),
            pltpu.SemaphoreType.DMA,
            pltpu.SemaphoreType.DMA,
        ],
    )
    def cnt(dst_a, dst_b, zacc, ones_hbm, out_a, out_b,
            didx_v, rows_v, acc_sh, ss0, ss1):
        core = lax.axis_index("c")
        tile = lax.axis_index("s")
        r0 = tile * ROWS_PER_TILE

        pltpu.sync_copy(zacc.at[pl.ds(r0, ROWS_PER_TILE)],
                        acc_sh.at[pl.ds(r0, ROWS_PER_TILE)])
        pltpu.sync_copy(ones_hbm, rows_v)
        plsc.subcore_barrier()

        def loop(dst2d):
            for bi in range(SEG_NBLK):
                brow = tile * SEG_NCH + bi * SEG_IDXB
                pltpu.sync_copy(dst2d.at[pl.ds(brow, SEG_IDXB)], didx_v)

                def pair(p, carry):
                    s0 = pltpu.async_copy(rows_v.at[pl.ds(0, SEGC)],
                                          acc_sh.at[didx_v.at[2 * p]],
                                          ss0, add=True)
                    s1 = pltpu.async_copy(rows_v.at[pl.ds(0, SEGC)],
                                          acc_sh.at[didx_v.at[2 * p + 1]],
                                          ss1, add=True)
                    s0.wait()
                    s1.wait()
                    return carry

                lax.fori_loop(0, SEG_IDXB // 2, pair, 0)

        @pl.when(core == 0)
        def _():
            loop(dst_a)

        @pl.when(core == 1)
        def _():
            loop(dst_b)

        plsc.subcore_barrier()

        def out_copy(c_out):
            ro = 0
            while ro < ROWS_PER_TILE:
                rn = min(CHUNK, ROWS_PER_TILE - ro)
                pltpu.sync_copy(acc_sh.at[pl.ds(r0 + ro, rn)],
                                rows_v.at[pl.ds(0, rn)])
                pltpu.sync_copy(rows_v.at[pl.ds(0, rn)],
                                c_out.at[pl.ds(r0 + ro, rn)])
                ro += rn

        @pl.when(core == 0)
        def _():
            out_copy(out_a)

        @pl.when(core == 1)
        def _():
            out_copy(out_b)

    return cnt


_cnt_kernel = _make_cnt_kernel()


def _cls_gather_body(ou_hbm, oc_hbm, ia2d, ib2d, ga_out, gb_out,
                     iav, ibv, a0, a1, b0, b1,
                     ga0, ga1, gb0, gb1, wa0, wa1, wb0, wb1):
    core = lax.axis_index("c")
    tile = lax.axis_index("s")
    wid = core * SC_TILES + tile
    ebase = wid * EL_PER_TILE
    npair = N_CHUNKS_CLS // 2

    pltpu.sync_copy(ia2d.at[wid], iav)
    pltpu.sync_copy(ib2d.at[wid], ibv)

    pltpu.async_copy(ou_hbm.at[iav.at[0]], a0, ga0)
    pltpu.async_copy(oc_hbm.at[ibv.at[0]], b0, gb0)
    pltpu.async_copy(ou_hbm.at[iav.at[1]], a1, ga1)
    pltpu.async_copy(oc_hbm.at[ibv.at[1]], b1, gb1)

    def pair(p, carry):
        j0 = 2 * p
        j1 = 2 * p + 1
        o0 = ebase + j0 * CHUNK
        o1 = ebase + j1 * CHUNK
        pltpu.make_async_copy(ou_hbm.at[iav.at[j0]], a0, ga0).wait()
        w_a0 = pltpu.async_copy(a0, ga_out.at[pl.ds(o0, CHUNK)], wa0)
        pltpu.make_async_copy(oc_hbm.at[ibv.at[j0]], b0, gb0).wait()
        w_b0 = pltpu.async_copy(b0, gb_out.at[pl.ds(o0, CHUNK)], wb0)
        pltpu.make_async_copy(ou_hbm.at[iav.at[j1]], a1, ga1).wait()
        w_a1 = pltpu.async_copy(a1, ga_out.at[pl.ds(o1, CHUNK)], wa1)
        pltpu.make_async_copy(oc_hbm.at[ibv.at[j1]], b1, gb1).wait()
        w_b1 = pltpu.async_copy(b1, gb_out.at[pl.ds(o1, CHUNK)], wb1)

        @pl.when(p < npair - 1)
        def _():
            w_a0.wait()
            pltpu.async_copy(ou_hbm.at[iav.at[j0 + 2]], a0, ga0)
            w_b0.wait()
            pltpu.async_copy(oc_hbm.at[ibv.at[j0 + 2]], b0, gb0)
            w_a1.wait()
            pltpu.async_copy(ou_hbm.at[iav.at[j1 + 2]], a1, ga1)
            w_b1.wait()
            pltpu.async_copy(oc_hbm.at[ibv.at[j1 + 2]], b1, gb1)

        @pl.when(p == npair - 1)
        def _():
            w_a0.wait()
            w_b0.wait()
            w_a1.wait()
            w_b1.wait()

        return carry

    lax.fori_loop(0, npair, pair, 0)


_cls_gather = pl.kernel(
    _cls_gather_body,
    out_type=[jax.ShapeDtypeStruct((EL_PAD, H), jnp.float32)] * 2,
    mesh=plsc.VectorSubcoreMesh(core_axis_name="c", subcore_axis_name="s"),
    scratch_types=(
        [pltpu.VMEM((32, CHUNK), jnp.int32)] * 2 +
        [pltpu.VMEM((CHUNK, H), jnp.float32)] * 4 +
        [pltpu.SemaphoreType.DMA] * 8
    ),
)

CLS_MB = 1024


def _cls_dot_body(a_ref, b_ref, o_ref):
    o_ref[...] = jnp.sum(a_ref[...] * b_ref[...], axis=1, keepdims=True)


def _cls_dot(ga, gb):
    grid = (EL_PAD // CLS_MB,)
    row = pl.BlockSpec((CLS_MB, H), lambda i: (i, 0))
    out = pl.BlockSpec((CLS_MB, 1), lambda i: (i, 0))
    return pl.pallas_call(
        _cls_dot_body,
        grid=grid,
        in_specs=[row, row],
        out_specs=out,
        out_shape=jax.ShapeDtypeStruct((EL_PAD, 1), jnp.float32),
    )(ga, gb)


def _pad_edges(idx, n, lo, hi):
    m = n - idx.shape[0]
    pad = lo + jnp.arange(m, dtype=jnp.int32) % (hi - lo)
    return jnp.concatenate([idx.astype(jnp.int32), pad])


def kernel(x_user, x_content, user_lin_w, user_lin_b, content_lin_w,
           content_lin_b, user_emb, content_emb,
           c1_uc_wl, c1_uc_bl, c1_uc_wr, c1_cu_wl, c1_cu_bl, c1_cu_wr,
           c2_uc_wl, c2_uc_bl, c2_uc_wr, c2_cu_wl, c2_cu_bl, c2_cu_wr,
           edge_index_uc, edge_index_cu, edge_label_index):
    n2d = E_PAD // SEGC
    ndum = N_PAD - DUMMY_ROW
    src_cu = _pad_edges(edge_index_cu[0], E_PAD, 0, NU).reshape(n2d, SEGC)
    dst_cu = _pad_edges(edge_index_cu[1], E_PAD, DUMMY_ROW,
                        DUMMY_ROW + ndum).reshape(n2d, SEGC)
    src_uc = _pad_edges(edge_index_uc[0], E_PAD, 0, NC).reshape(n2d, SEGC)
    dst_uc = _pad_edges(edge_index_uc[1], E_PAD, DUMMY_ROW,
                        DUMMY_ROW + ndum).reshape(n2d, SEGC)
    def _labels3d(idx):
        x = _pad_edges(idx, EL_PAD, 0, NU).reshape(32, N_CHUNKS_CLS, CHUNK)
        return jnp.pad(x, ((0, 0), (0, 32 - N_CHUNKS_CLS), (0, 0)))

    la = _labels3d(edge_label_index[0])
    lb = _labels3d(edge_label_index[1])

    zacc = jnp.zeros((N_PAD, H), jnp.float32)
    ones_hbm = jnp.ones((CHUNK, H), jnp.float32)

    cnt_u, cnt_c = _cnt_kernel(dst_cu, dst_uc, zacc, ones_hbm)

    yu1, ru1 = _phase_a(x_user, user_lin_w, user_lin_b, user_emb,
                        c1_uc_wl, c1_cu_wr, c1_cu_bl)
    yc1, rc1 = _phase_a(x_content, content_lin_w, content_lin_b, content_emb,
                        c1_cu_wl, c1_uc_wr, c1_uc_bl)

    su1, sc1 = _seg_kernel(yc1, yu1, src_cu, dst_cu, src_uc, dst_uc, zacc)

    yu2, ru2 = _phase_c(su1, cnt_u, ru1, c2_uc_wl, c2_cu_wr, c2_cu_bl)
    yc2, rc2 = _phase_c(sc1, cnt_c, rc1, c2_cu_wl, c2_uc_wr, c2_uc_bl)

    su2, sc2 = _seg_kernel(yc2, yu2, src_cu, dst_cu, src_uc, dst_uc, zacc)

    ou = _phase_e(su2, cnt_u, ru2)
    oc = _phase_e(sc2, cnt_c, rc2)

    ga, gb = _cls_gather(ou, oc, la, lb)
    pred = _cls_dot(ga, gb)
    return pred[:EL, 0]


# speedup vs baseline: 2.1746x; 1.0139x over previous
import functools

import jax
import jax.numpy as jnp
from jax import lax
from jax.experimental import pallas as pl
from jax.experimental.pallas import tpu as pltpu
from jax.experimental.pallas import tpu_sc as plsc

NU = 10000
NC = 10000
E = 320000
EL = 100000
F = 128
H = 128

SC_CORES = 2
SC_TILES = 16

CHUNK = 128
SEGC = 32
E_PER_TILE = 20480
E_PAD = E_PER_TILE * SC_TILES
N_CHUNKS = E_PER_TILE // CHUNK
SEG_NCH = E_PER_TILE // SEGC
SEG_IDXB = 64
SEG_NBLK = SEG_NCH // SEG_IDXB
ROWS_PER_TILE = 632
N_PAD = ROWS_PER_TILE * SC_TILES
DUMMY_ROW = 10000

N_CHUNKS_CLS = 26
EL_PER_TILE = N_CHUNKS_CLS * CHUNK
EL_PAD = EL_PER_TILE * SC_CORES * SC_TILES

MB = 1000


def _dot_t(a, b):
    return lax.dot_general(a, b, (((1,), (1,)), ((), ())),
                           preferred_element_type=jnp.float32)


def _phase_a_body(x_ref, w1_ref, b1_ref, emb_ref, wl_ref, wr_ref, bl_ref,
                  y_ref, r_ref):
    t = _dot_t(x_ref[...], w1_ref[...]) + b1_ref[...] + emb_ref[...]
    y_ref[...] = _dot_t(t, wl_ref[...])
    r_ref[...] = _dot_t(t, wr_ref[...]) + bl_ref[...]


def _phase_a(x, w1, b1, emb, wl, wr, bl):
    grid = (NU // MB,)
    row = pl.BlockSpec((MB, F), lambda i: (i, 0))
    full = pl.BlockSpec((H, F), lambda i: (0, 0))
    vec = pl.BlockSpec((1, H), lambda i: (0, 0))
    return pl.pallas_call(
        _phase_a_body,
        grid=grid,
        in_specs=[row, full, vec, row, full, full, vec],
        out_specs=[row, row],
        out_shape=[jax.ShapeDtypeStruct((NU, H), jnp.float32)] * 2,
    )(x, w1, b1.reshape(1, H), emb, wl, wr, bl.reshape(1, H))


def _phase_c_body(s_ref, cnt_ref, r1_ref, wl_ref, wr_ref, bl_ref,
                  y2_ref, r2_ref):
    cnt = jnp.maximum(cnt_ref[...][:, 0:1], 1.0)
    h = jnp.maximum(s_ref[...] / cnt + r1_ref[...], 0.0)
    y2_ref[...] = _dot_t(h, wl_ref[...])
    r2_ref[...] = _dot_t(h, wr_ref[...]) + bl_ref[...]


def _phase_c(s, cnt, r1, wl, wr, bl):
    grid = (NU // MB,)
    row = pl.BlockSpec((MB, H), lambda i: (i, 0))
    crow = pl.BlockSpec((MB, H), lambda i: (i, 0))
    full = pl.BlockSpec((H, H), lambda i: (0, 0))
    vec = pl.BlockSpec((1, H), lambda i: (0, 0))
    return pl.pallas_call(
        _phase_c_body,
        grid=grid,
        in_specs=[row, crow, row, full, full, vec],
        out_specs=[row, row],
        out_shape=[jax.ShapeDtypeStruct((NU, H), jnp.float32)] * 2,
    )(s, cnt, r1, wl, wr, bl.reshape(1, H))


def _phase_e_body(s_ref, cnt_ref, r2_ref, o_ref):
    cnt = jnp.maximum(cnt_ref[...][:, 0:1], 1.0)
    o_ref[...] = s_ref[...] / cnt + r2_ref[...]


def _phase_e(s, cnt, r2):
    grid = (NU // MB,)
    row = pl.BlockSpec((MB, H), lambda i: (i, 0))
    crow = pl.BlockSpec((MB, H), lambda i: (i, 0))
    return pl.pallas_call(
        _phase_e_body,
        grid=grid,
        in_specs=[row, crow, row],
        out_specs=row,
        out_shape=jax.ShapeDtypeStruct((NU, H), jnp.float32),
    )(s, cnt, r2)


IDXB = 16
NBLK = N_CHUNKS // IDXB


def _make_seg_kernel():
    mesh = plsc.VectorSubcoreMesh(core_axis_name="c", subcore_axis_name="s")

    @functools.partial(
        pl.kernel,
        out_type=[jax.ShapeDtypeStruct((N_PAD, H), jnp.float32)] * 2,
        mesh=mesh,
        scratch_types=(
            [pltpu.VMEM((SEG_IDXB, SEGC), jnp.int32)] * 2 +
            [pltpu.VMEM((SEGC, H), jnp.float32)] * 8 +
            [pltpu.VMEM_SHARED((N_PAD, H), jnp.float32)] +
            [pltpu.SemaphoreType.DMA] * 16
        ),
    )
    def seg(y_a, y_b, src_a, dst_a, src_b, dst_b, zacc,
            out_a, out_b, sidx_v, didx_v,
            r0_, r1_, r2_, r3_, r4_, r5_, r6_, r7_,
            acc_sh, g0s, g1s, g2s, g3s, g4s, g5s, g6s, g7s,
            s0s, s1s, s2s, s3s, s4s, s5s, s6s, s7s):
        core = lax.axis_index("c")
        tile = lax.axis_index("s")
        r0 = tile * ROWS_PER_TILE

        pltpu.sync_copy(zacc.at[pl.ds(r0, ROWS_PER_TILE)],
                        acc_sh.at[pl.ds(r0, ROWS_PER_TILE)])
        plsc.subcore_barrier()

        bufs = [(r0_, g0s, s0s), (r1_, g1s, s1s), (r2_, g2s, s2s),
                (r3_, g3s, s3s), (r4_, g4s, s4s), (r5_, g5s, s5s),
                (r6_, g6s, s6s), (r7_, g7s, s7s)]

        def loop(y_hbm, src2d, dst2d):
            for bi in range(SEG_NBLK):
                brow = tile * SEG_NCH + bi * SEG_IDXB
                pltpu.sync_copy(src2d.at[pl.ds(brow, SEG_IDXB)], sidx_v)
                pltpu.sync_copy(dst2d.at[pl.ds(brow, SEG_IDXB)], didx_v)

                def octet(q, carry):
                    jb = 8 * q
                    gs = [pltpu.async_copy(y_hbm.at[sidx_v.at[jb + k]],
                                           bufs[k][0], bufs[k][1])
                          for k in range(8)]
                    ss = []
                    for k in range(8):
                        gs[k].wait()
                        ss.append(pltpu.async_copy(
                            bufs[k][0], acc_sh.at[didx_v.at[jb + k]],
                            bufs[k][2], add=True))
                    for k in range(8):
                        ss[k].wait()
                    return carry

                lax.fori_loop(0, SEG_IDXB // 8, octet, 0)

        @pl.when(core == 0)
        def _():
            loop(y_a, src_a, dst_a)

        @pl.when(core == 1)
        def _():
            loop(y_b, src_b, dst_b)

        plsc.subcore_barrier()

        def out_copy(s_out):
            ro = 0
            while ro < ROWS_PER_TILE:
                rn = min(SEGC, ROWS_PER_TILE - ro)
                pltpu.sync_copy(acc_sh.at[pl.ds(r0 + ro, rn)],
                                r0_.at[pl.ds(0, rn)])
                pltpu.sync_copy(r0_.at[pl.ds(0, rn)],
                                s_out.at[pl.ds(r0 + ro, rn)])
                ro += rn

        @pl.when(core == 0)
        def _():
            out_copy(out_a)

        @pl.when(core == 1)
        def _():
            out_copy(out_b)

    return seg


_seg_kernel = _make_seg_kernel()


def _make_cnt_kernel():
    mesh = plsc.VectorSubcoreMesh(core_axis_name="c", subcore_axis_name="s")

    @functools.partial(
        pl.kernel,
        out_type=[jax.ShapeDtypeStruct((N_PAD, H), jnp.float32)] * 2,
        mesh=mesh,
        scratch_types=[
            pltpu.VMEM((SEG_IDXB, SEGC), jnp.int32),
            pltpu.VMEM((CHUNK, H), jnp.float32),
            pltpu.VMEM_SHARED((N_PAD, H), jnp.float32),
            pltpu.SemaphoreType.DMA,
            pltpu.SemaphoreType.DMA,
            pltpu.SemaphoreType.DMA,
            pltpu.SemaphoreType.DMA,
        ],
    )
    def cnt(dst_a, dst_b, zacc, ones_hbm, out_a, out_b,
            didx_v, rows_v, acc_sh, ss0, ss1, ss2, ss3):
        core = lax.axis_index("c")
        tile = lax.axis_index("s")
        r0 = tile * ROWS_PER_TILE

        pltpu.sync_copy(zacc.at[pl.ds(r0, ROWS_PER_TILE)],
                        acc_sh.at[pl.ds(r0, ROWS_PER_TILE)])
        pltpu.sync_copy(ones_hbm, rows_v)
        plsc.subcore_barrier()

        def loop(dst2d):
            for bi in range(SEG_NBLK):
                brow = tile * SEG_NCH + bi * SEG_IDXB
                pltpu.sync_copy(dst2d.at[pl.ds(brow, SEG_IDXB)], didx_v)

                def quad(q, carry):
                    ss = [pltpu.async_copy(rows_v.at[pl.ds(0, SEGC)],
                                           acc_sh.at[didx_v.at[4 * q + k]],
                                           [ss0, ss1, ss2, ss3][k], add=True)
                          for k in range(4)]
                    for s in ss:
                        s.wait()
                    return carry

                lax.fori_loop(0, SEG_IDXB // 4, quad, 0)

        @pl.when(core == 0)
        def _():
            loop(dst_a)

        @pl.when(core == 1)
        def _():
            loop(dst_b)

        plsc.subcore_barrier()

        def out_copy(c_out):
            ro = 0
            while ro < ROWS_PER_TILE:
                rn = min(CHUNK, ROWS_PER_TILE - ro)
                pltpu.sync_copy(acc_sh.at[pl.ds(r0 + ro, rn)],
                                rows_v.at[pl.ds(0, rn)])
                pltpu.sync_copy(rows_v.at[pl.ds(0, rn)],
                                c_out.at[pl.ds(r0 + ro, rn)])
                ro += rn

        @pl.when(core == 0)
        def _():
            out_copy(out_a)

        @pl.when(core == 1)
        def _():
            out_copy(out_b)

    return cnt


_cnt_kernel = _make_cnt_kernel()


def _cls_gather_body(ou_hbm, oc_hbm, ia2d, ib2d, ga_out, gb_out,
                     iav, ibv, a0, a1, b0, b1,
                     ga0, ga1, gb0, gb1, wa0, wa1, wb0, wb1):
    core = lax.axis_index("c")
    tile = lax.axis_index("s")
    wid = core * SC_TILES + tile
    ebase = wid * EL_PER_TILE
    npair = N_CHUNKS_CLS // 2

    pltpu.sync_copy(ia2d.at[wid], iav)
    pltpu.sync_copy(ib2d.at[wid], ibv)

    pltpu.async_copy(ou_hbm.at[iav.at[0]], a0, ga0)
    pltpu.async_copy(oc_hbm.at[ibv.at[0]], b0, gb0)
    pltpu.async_copy(ou_hbm.at[iav.at[1]], a1, ga1)
    pltpu.async_copy(oc_hbm.at[ibv.at[1]], b1, gb1)

    def pair(p, carry):
        j0 = 2 * p
        j1 = 2 * p + 1
        o0 = ebase + j0 * CHUNK
        o1 = ebase + j1 * CHUNK
        pltpu.make_async_copy(ou_hbm.at[iav.at[j0]], a0, ga0).wait()
        w_a0 = pltpu.async_copy(a0, ga_out.at[pl.ds(o0, CHUNK)], wa0)
        pltpu.make_async_copy(oc_hbm.at[ibv.at[j0]], b0, gb0).wait()
        w_b0 = pltpu.async_copy(b0, gb_out.at[pl.ds(o0, CHUNK)], wb0)
        pltpu.make_async_copy(ou_hbm.at[iav.at[j1]], a1, ga1).wait()
        w_a1 = pltpu.async_copy(a1, ga_out.at[pl.ds(o1, CHUNK)], wa1)
        pltpu.make_async_copy(oc_hbm.at[ibv.at[j1]], b1, gb1).wait()
        w_b1 = pltpu.async_copy(b1, gb_out.at[pl.ds(o1, CHUNK)], wb1)

        @pl.when(p < npair - 1)
        def _():
            w_a0.wait()
            pltpu.async_copy(ou_hbm.at[iav.at[j0 + 2]], a0, ga0)
            w_b0.wait()
            pltpu.async_copy(oc_hbm.at[ibv.at[j0 + 2]], b0, gb0)
            w_a1.wait()
            pltpu.async_copy(ou_hbm.at[iav.at[j1 + 2]], a1, ga1)
            w_b1.wait()
            pltpu.async_copy(oc_hbm.at[ibv.at[j1 + 2]], b1, gb1)

        @pl.when(p == npair - 1)
        def _():
            w_a0.wait()
            w_b0.wait()
            w_a1.wait()
            w_b1.wait()

        return carry

    lax.fori_loop(0, npair, pair, 0)


_cls_gather = pl.kernel(
    _cls_gather_body,
    out_type=[jax.ShapeDtypeStruct((EL_PAD, H), jnp.float32)] * 2,
    mesh=plsc.VectorSubcoreMesh(core_axis_name="c", subcore_axis_name="s"),
    scratch_types=(
        [pltpu.VMEM((32, CHUNK), jnp.int32)] * 2 +
        [pltpu.VMEM((CHUNK, H), jnp.float32)] * 4 +
        [pltpu.SemaphoreType.DMA] * 8
    ),
)

CLS_MB = 1024


def _cls_dot_body(a_ref, b_ref, o_ref):
    o_ref[...] = jnp.sum(a_ref[...] * b_ref[...], axis=1, keepdims=True)


def _cls_dot(ga, gb):
    grid = (EL_PAD // CLS_MB,)
    row = pl.BlockSpec((CLS_MB, H), lambda i: (i, 0))
    out = pl.BlockSpec((CLS_MB, 1), lambda i: (i, 0))
    return pl.pallas_call(
        _cls_dot_body,
        grid=grid,
        in_specs=[row, row],
        out_specs=out,
        out_shape=jax.ShapeDtypeStruct((EL_PAD, 1), jnp.float32),
    )(ga, gb)


def _pad_edges(idx, n, lo, hi):
    m = n - idx.shape[0]
    pad = lo + jnp.arange(m, dtype=jnp.int32) % (hi - lo)
    return jnp.concatenate([idx.astype(jnp.int32), pad])


def kernel(x_user, x_content, user_lin_w, user_lin_b, content_lin_w,
           content_lin_b, user_emb, content_emb,
           c1_uc_wl, c1_uc_bl, c1_uc_wr, c1_cu_wl, c1_cu_bl, c1_cu_wr,
           c2_uc_wl, c2_uc_bl, c2_uc_wr, c2_cu_wl, c2_cu_bl, c2_cu_wr,
           edge_index_uc, edge_index_cu, edge_label_index):
    n2d = E_PAD // SEGC
    ndum = N_PAD - DUMMY_ROW
    src_cu = _pad_edges(edge_index_cu[0], E_PAD, 0, NU).reshape(n2d, SEGC)
    dst_cu = _pad_edges(edge_index_cu[1], E_PAD, DUMMY_ROW,
                        DUMMY_ROW + ndum).reshape(n2d, SEGC)
    src_uc = _pad_edges(edge_index_uc[0], E_PAD, 0, NC).reshape(n2d, SEGC)
    dst_uc = _pad_edges(edge_index_uc[1], E_PAD, DUMMY_ROW,
                        DUMMY_ROW + ndum).reshape(n2d, SEGC)
    def _labels3d(idx):
        x = _pad_edges(idx, EL_PAD, 0, NU).reshape(32, N_CHUNKS_CLS, CHUNK)
        return jnp.pad(x, ((0, 0), (0, 32 - N_CHUNKS_CLS), (0, 0)))

    la = _labels3d(edge_label_index[0])
    lb = _labels3d(edge_label_index[1])

    zacc = jnp.zeros((N_PAD, H), jnp.float32)
    ones_hbm = jnp.ones((CHUNK, H), jnp.float32)

    cnt_u, cnt_c = _cnt_kernel(dst_cu, dst_uc, zacc, ones_hbm)

    yu1, ru1 = _phase_a(x_user, user_lin_w, user_lin_b, user_emb,
                        c1_uc_wl, c1_cu_wr, c1_cu_bl)
    yc1, rc1 = _phase_a(x_content, content_lin_w, content_lin_b, content_emb,
                        c1_cu_wl, c1_uc_wr, c1_uc_bl)

    su1, sc1 = _seg_kernel(yc1, yu1, src_cu, dst_cu, src_uc, dst_uc, zacc)

    yu2, ru2 = _phase_c(su1, cnt_u, ru1, c2_uc_wl, c2_cu_wr, c2_cu_bl)
    yc2, rc2 = _phase_c(sc1, cnt_c, rc1, c2_cu_wl, c2_uc_wr, c2_uc_bl)

    su2, sc2 = _seg_kernel(yc2, yu2, src_cu, dst_cu, src_uc, dst_uc, zacc)

    ou = _phase_e(su2, cnt_u, ru2)
    oc = _phase_e(sc2, cnt_c, rc2)

    ga, gb = _cls_gather(ou, oc, la, lb)
    pred = _cls_dot(ga, gb)
    return pred[:EL, 0]


# speedup vs baseline: 2.1779x; 1.0015x over previous
import functools

import jax
import jax.numpy as jnp
from jax import lax
from jax.experimental import pallas as pl
from jax.experimental.pallas import tpu as pltpu
from jax.experimental.pallas import tpu_sc as plsc

NU = 10000
NC = 10000
E = 320000
EL = 100000
F = 128
H = 128

SC_CORES = 2
SC_TILES = 16

CHUNK = 128
SEGC = 32
E_PER_TILE = 20480
E_PAD = E_PER_TILE * SC_TILES
SEG_NCH = E_PER_TILE // SEGC
SEG_IDXB = 64
SEG_NBLK = SEG_NCH // SEG_IDXB
ROWS_PER_TILE = 632
N_PAD = ROWS_PER_TILE * SC_TILES
DUMMY_ROW = 10000

N_CHUNKS_CLS = 26
EL_PER_TILE = N_CHUNKS_CLS * CHUNK
EL_PAD = EL_PER_TILE * SC_CORES * SC_TILES

MB = 1000


def _dot_t(a, b):
    return lax.dot_general(a, b, (((1,), (1,)), ((), ())),
                           preferred_element_type=jnp.float32)


def _phase_a_body(x_ref, w1_ref, b1_ref, emb_ref, wl_ref, wr_ref, bl_ref,
                  y_ref, r_ref):
    t = _dot_t(x_ref[...], w1_ref[...]) + b1_ref[...] + emb_ref[...]
    y_ref[...] = _dot_t(t, wl_ref[...])
    r_ref[...] = _dot_t(t, wr_ref[...]) + bl_ref[...]


def _phase_a(x, w1, b1, emb, wl, wr, bl):
    grid = (NU // MB,)
    row = pl.BlockSpec((MB, F), lambda i: (i, 0))
    full = pl.BlockSpec((H, F), lambda i: (0, 0))
    vec = pl.BlockSpec((1, H), lambda i: (0, 0))
    return pl.pallas_call(
        _phase_a_body,
        grid=grid,
        in_specs=[row, full, vec, row, full, full, vec],
        out_specs=[row, row],
        out_shape=[jax.ShapeDtypeStruct((NU, H), jnp.float32)] * 2,
    )(x, w1, b1.reshape(1, H), emb, wl, wr, bl.reshape(1, H))


def _phase_c_body(s_ref, cnt_ref, r1_ref, wl_ref, wr_ref, bl_ref,
                  y2_ref, r2_ref):
    cnt = jnp.maximum(cnt_ref[...][:, 0:1], 1.0)
    h = jnp.maximum(s_ref[...] / cnt + r1_ref[...], 0.0)
    y2_ref[...] = _dot_t(h, wl_ref[...])
    r2_ref[...] = _dot_t(h, wr_ref[...]) + bl_ref[...]


def _phase_c(s, cnt, r1, wl, wr, bl):
    grid = (NU // MB,)
    row = pl.BlockSpec((MB, H), lambda i: (i, 0))
    crow = pl.BlockSpec((MB, H), lambda i: (i, 0))
    full = pl.BlockSpec((H, H), lambda i: (0, 0))
    vec = pl.BlockSpec((1, H), lambda i: (0, 0))
    return pl.pallas_call(
        _phase_c_body,
        grid=grid,
        in_specs=[row, crow, row, full, full, vec],
        out_specs=[row, row],
        out_shape=[jax.ShapeDtypeStruct((NU, H), jnp.float32)] * 2,
    )(s, cnt, r1, wl, wr, bl.reshape(1, H))


def _phase_e_body(s_ref, cnt_ref, r2_ref, o_ref):
    cnt = jnp.maximum(cnt_ref[...][:, 0:1], 1.0)
    o_ref[...] = s_ref[...] / cnt + r2_ref[...]


def _phase_e(s, cnt, r2):
    grid = (NU // MB,)
    row = pl.BlockSpec((MB, H), lambda i: (i, 0))
    crow = pl.BlockSpec((MB, H), lambda i: (i, 0))
    return pl.pallas_call(
        _phase_e_body,
        grid=grid,
        in_specs=[row, crow, row],
        out_specs=row,
        out_shape=jax.ShapeDtypeStruct((NU, H), jnp.float32),
    )(s, cnt, r2)


def _make_seg_kernel():
    mesh = plsc.VectorSubcoreMesh(core_axis_name="c", subcore_axis_name="s")

    @functools.partial(
        pl.kernel,
        out_type=[jax.ShapeDtypeStruct((N_PAD, H), jnp.float32)] * 2,
        mesh=mesh,
        scratch_types=(
            [pltpu.VMEM((SEG_IDXB, SEGC), jnp.int32)] * 2 +
            [pltpu.VMEM((SEGC, H), jnp.float32)] * 8 +
            [pltpu.VMEM_SHARED((N_PAD, H), jnp.float32)] +
            [pltpu.SemaphoreType.DMA] * 16
        ),
    )
    def seg(y_a, y_b, src_a, dst_a, src_b, dst_b, zacc,
            out_a, out_b, sidx_v, didx_v,
            r0_, r1_, r2_, r3_, r4_, r5_, r6_, r7_,
            acc_sh, g0s, g1s, g2s, g3s, g4s, g5s, g6s, g7s,
            s0s, s1s, s2s, s3s, s4s, s5s, s6s, s7s):
        core = lax.axis_index("c")
        tile = lax.axis_index("s")
        r0 = tile * ROWS_PER_TILE

        pltpu.sync_copy(zacc.at[pl.ds(r0, ROWS_PER_TILE)],
                        acc_sh.at[pl.ds(r0, ROWS_PER_TILE)])
        plsc.subcore_barrier()

        bufs = [(r0_, g0s, s0s), (r1_, g1s, s1s), (r2_, g2s, s2s),
                (r3_, g3s, s3s), (r4_, g4s, s4s), (r5_, g5s, s5s),
                (r6_, g6s, s6s), (r7_, g7s, s7s)]

        def loop(y_hbm, src2d, dst2d):
            for bi in range(SEG_NBLK):
                brow = tile * SEG_NCH + bi * SEG_IDXB
                pltpu.sync_copy(src2d.at[pl.ds(brow, SEG_IDXB)], sidx_v)
                pltpu.sync_copy(dst2d.at[pl.ds(brow, SEG_IDXB)], didx_v)

                def octet(q, carry):
                    jb = 8 * q
                    gs = [pltpu.async_copy(y_hbm.at[sidx_v.at[jb + k]],
                                           bufs[k][0], bufs[k][1])
                          for k in range(8)]
                    ss = []
                    for k in range(8):
                        gs[k].wait()
                        ss.append(pltpu.async_copy(
                            bufs[k][0], acc_sh.at[didx_v.at[jb + k]],
                            bufs[k][2], add=True))
                    for k in range(8):
                        ss[k].wait()
                    return carry

                lax.fori_loop(0, SEG_IDXB // 8, octet, 0)

        @pl.when(core == 0)
        def _():
            loop(y_a, src_a, dst_a)

        @pl.when(core == 1)
        def _():
            loop(y_b, src_b, dst_b)

        plsc.subcore_barrier()

        def out_copy(s_out):
            ro = 0
            while ro < ROWS_PER_TILE:
                rn = min(SEGC, ROWS_PER_TILE - ro)
                pltpu.sync_copy(acc_sh.at[pl.ds(r0 + ro, rn)],
                                r0_.at[pl.ds(0, rn)])
                pltpu.sync_copy(r0_.at[pl.ds(0, rn)],
                                s_out.at[pl.ds(r0 + ro, rn)])
                ro += rn

        @pl.when(core == 0)
        def _():
            out_copy(out_a)

        @pl.when(core == 1)
        def _():
            out_copy(out_b)

    return seg


_seg_kernel = _make_seg_kernel()


def _make_cnt_kernel():
    mesh = plsc.VectorSubcoreMesh(core_axis_name="c", subcore_axis_name="s")

    @functools.partial(
        pl.kernel,
        out_type=[jax.ShapeDtypeStruct((N_PAD, H), jnp.float32)] * 2,
        mesh=mesh,
        scratch_types=[
            pltpu.VMEM((SEG_IDXB, SEGC), jnp.int32),
            pltpu.VMEM((CHUNK, H), jnp.float32),
            pltpu.VMEM_SHARED((N_PAD, H), jnp.float32),
            pltpu.SemaphoreType.DMA,
            pltpu.SemaphoreType.DMA,
            pltpu.SemaphoreType.DMA,
            pltpu.SemaphoreType.DMA,
        ],
    )
    def cnt(dst_a, dst_b, zacc, ones_hbm, out_a, out_b,
            didx_v, rows_v, acc_sh, ss0, ss1, ss2, ss3):
        core = lax.axis_index("c")
        tile = lax.axis_index("s")
        r0 = tile * ROWS_PER_TILE

        pltpu.sync_copy(zacc.at[pl.ds(r0, ROWS_PER_TILE)],
                        acc_sh.at[pl.ds(r0, ROWS_PER_TILE)])
        pltpu.sync_copy(ones_hbm, rows_v)
        plsc.subcore_barrier()

        def loop(dst2d):
            for bi in range(SEG_NBLK):
                brow = tile * SEG_NCH + bi * SEG_IDXB
                pltpu.sync_copy(dst2d.at[pl.ds(brow, SEG_IDXB)], didx_v)

                def quad(q, carry):
                    ss = [pltpu.async_copy(rows_v.at[pl.ds(0, SEGC)],
                                           acc_sh.at[didx_v.at[4 * q + k]],
                                           [ss0, ss1, ss2, ss3][k], add=True)
                          for k in range(4)]
                    for s in ss:
                        s.wait()
                    return carry

                lax.fori_loop(0, SEG_IDXB // 4, quad, 0)

        @pl.when(core == 0)
        def _():
            loop(dst_a)

        @pl.when(core == 1)
        def _():
            loop(dst_b)

        plsc.subcore_barrier()

        def out_copy(c_out):
            ro = 0
            while ro < ROWS_PER_TILE:
                rn = min(CHUNK, ROWS_PER_TILE - ro)
                pltpu.sync_copy(acc_sh.at[pl.ds(r0 + ro, rn)],
                                rows_v.at[pl.ds(0, rn)])
                pltpu.sync_copy(rows_v.at[pl.ds(0, rn)],
                                c_out.at[pl.ds(r0 + ro, rn)])
                ro += rn

        @pl.when(core == 0)
        def _():
            out_copy(out_a)

        @pl.when(core == 1)
        def _():
            out_copy(out_b)

    return cnt


_cnt_kernel = _make_cnt_kernel()


def _cls_gather_body(ou_hbm, oc_hbm, ia2d, ib2d, ga_out, gb_out,
                     iav, ibv, a0, a1, b0, b1,
                     ga0, ga1, gb0, gb1, wa0, wa1, wb0, wb1):
    core = lax.axis_index("c")
    tile = lax.axis_index("s")
    wid = core * SC_TILES + tile
    ebase = wid * EL_PER_TILE
    npair = N_CHUNKS_CLS // 2

    pltpu.sync_copy(ia2d.at[wid], iav)
    pltpu.sync_copy(ib2d.at[wid], ibv)

    pltpu.async_copy(ou_hbm.at[iav.at[0]], a0, ga0)
    pltpu.async_copy(oc_hbm.at[ibv.at[0]], b0, gb0)
    pltpu.async_copy(ou_hbm.at[iav.at[1]], a1, ga1)
    pltpu.async_copy(oc_hbm.at[ibv.at[1]], b1, gb1)

    def pair(p, carry):
        j0 = 2 * p
        j1 = 2 * p + 1
        o0 = ebase + j0 * CHUNK
        o1 = ebase + j1 * CHUNK
        pltpu.make_async_copy(ou_hbm.at[iav.at[j0]], a0, ga0).wait()
        w_a0 = pltpu.async_copy(a0, ga_out.at[pl.ds(o0, CHUNK)], wa0)
        pltpu.make_async_copy(oc_hbm.at[ibv.at[j0]], b0, gb0).wait()
        w_b0 = pltpu.async_copy(b0, gb_out.at[pl.ds(o0, CHUNK)], wb0)
        pltpu.make_async_copy(ou_hbm.at[iav.at[j1]], a1, ga1).wait()
        w_a1 = pltpu.async_copy(a1, ga_out.at[pl.ds(o1, CHUNK)], wa1)
        pltpu.make_async_copy(oc_hbm.at[ibv.at[j1]], b1, gb1).wait()
        w_b1 = pltpu.async_copy(b1, gb_out.at[pl.ds(o1, CHUNK)], wb1)

        @pl.when(p < npair - 1)
        def _():
            w_a0.wait()
            pltpu.async_copy(ou_hbm.at[iav.at[j0 + 2]], a0, ga0)
            w_b0.wait()
            pltpu.async_copy(oc_hbm.at[ibv.at[j0 + 2]], b0, gb0)
            w_a1.wait()
            pltpu.async_copy(ou_hbm.at[iav.at[j1 + 2]], a1, ga1)
            w_b1.wait()
            pltpu.async_copy(oc_hbm.at[ibv.at[j1 + 2]], b1, gb1)

        @pl.when(p == npair - 1)
        def _():
            w_a0.wait()
            w_b0.wait()
            w_a1.wait()
            w_b1.wait()

        return carry

    lax.fori_loop(0, npair, pair, 0)


_cls_gather = pl.kernel(
    _cls_gather_body,
    out_type=[jax.ShapeDtypeStruct((EL_PAD, H), jnp.float32)] * 2,
    mesh=plsc.VectorSubcoreMesh(core_axis_name="c", subcore_axis_name="s"),
    scratch_types=(
        [pltpu.VMEM((32, CHUNK), jnp.int32)] * 2 +
        [pltpu.VMEM((CHUNK, H), jnp.float32)] * 4 +
        [pltpu.SemaphoreType.DMA] * 8
    ),
)

CLS_MB = 1024


def _cls_dot_body(a_ref, b_ref, o_ref):
    o_ref[...] = jnp.sum(a_ref[...] * b_ref[...], axis=1, keepdims=True)


def _cls_dot(ga, gb):
    grid = (EL_PAD // CLS_MB,)
    row = pl.BlockSpec((CLS_MB, H), lambda i: (i, 0))
    out = pl.BlockSpec((CLS_MB, 1), lambda i: (i, 0))
    return pl.pallas_call(
        _cls_dot_body,
        grid=grid,
        in_specs=[row, row],
        out_specs=out,
        out_shape=jax.ShapeDtypeStruct((EL_PAD, 1), jnp.float32),
    )(ga, gb)


def _pad_edges(idx, n, lo, hi):
    m = n - idx.shape[0]
    pad = lo + jnp.arange(m, dtype=jnp.int32) % (hi - lo)
    return jnp.concatenate([idx.astype(jnp.int32), pad])


def kernel(x_user, x_content, user_lin_w, user_lin_b, content_lin_w,
           content_lin_b, user_emb, content_emb,
           c1_uc_wl, c1_uc_bl, c1_uc_wr, c1_cu_wl, c1_cu_bl, c1_cu_wr,
           c2_uc_wl, c2_uc_bl, c2_uc_wr, c2_cu_wl, c2_cu_bl, c2_cu_wr,
           edge_index_uc, edge_index_cu, edge_label_index):
    n2d = E_PAD // SEGC
    ndum = N_PAD - DUMMY_ROW
    src_cu = _pad_edges(edge_index_cu[0], E_PAD, 0, NU).reshape(n2d, SEGC)
    dst_cu = _pad_edges(edge_index_cu[1], E_PAD, DUMMY_ROW,
                        DUMMY_ROW + ndum).reshape(n2d, SEGC)
    src_uc = _pad_edges(edge_index_uc[0], E_PAD, 0, NC).reshape(n2d, SEGC)
    dst_uc = _pad_edges(edge_index_uc[1], E_PAD, DUMMY_ROW,
                        DUMMY_ROW + ndum).reshape(n2d, SEGC)
    def _labels3d(idx):
        x = _pad_edges(idx, EL_PAD, 0, NU).reshape(32, N_CHUNKS_CLS, CHUNK)
        return jnp.pad(x, ((0, 0), (0, 32 - N_CHUNKS_CLS), (0, 0)))

    la = _labels3d(edge_label_index[0])
    lb = _labels3d(edge_label_index[1])

    zacc = jnp.zeros((N_PAD, H), jnp.float32)
    ones_hbm = jnp.ones((CHUNK, H), jnp.float32)

    cnt_u, cnt_c = _cnt_kernel(dst_cu, dst_uc, zacc, ones_hbm)

    yu1, ru1 = _phase_a(x_user, user_lin_w, user_lin_b, user_emb,
                        c1_uc_wl, c1_cu_wr, c1_cu_bl)
    yc1, rc1 = _phase_a(x_content, content_lin_w, content_lin_b, content_emb,
                        c1_cu_wl, c1_uc_wr, c1_uc_bl)

    su1, sc1 = _seg_kernel(yc1, yu1, src_cu, dst_cu, src_uc, dst_uc, zacc)

    yu2, ru2 = _phase_c(su1, cnt_u, ru1, c2_uc_wl, c2_cu_wr, c2_cu_bl)
    yc2, rc2 = _phase_c(sc1, cnt_c, rc1, c2_cu_wl, c2_uc_wr, c2_uc_bl)

    su2, sc2 = _seg_kernel(yc2, yu2, src_cu, dst_cu, src_uc, dst_uc, zacc)

    ou = _phase_e(su2, cnt_u, ru2)
    oc = _phase_e(sc2, cnt_c, rc2)

    ga, gb = _cls_gather(ou, oc, la, lb)
    pred = _cls_dot(ga, gb)
    return pred[:EL, 0]
